# baseline (speedup 1.0000x reference)
import functools

import jax
import jax.numpy as jnp
from jax import lax
from jax.experimental import pallas as pl
from jax.experimental.pallas import tpu as pltpu
from jax.experimental.pallas import tpu_sc as plsc

B = 16384
L = 20
D = 128
D_OUT = 2 * D

NUM_WORKERS = 32
ROWS_PER_W = B // NUM_WORKERS
CHUNK = 16
N_CHUNKS = ROWS_PER_W // CHUNK
LANES = 16
CL = CHUNK * L
NOUT = 4
UNROLL = 4


def _body(titles_hbm, tokens_hbm, title_tab, text_tab, out_hbm,
          tokbuf0, tokbuf1, tidx0, tidx1, tokrows0, tokrows1,
          outbuf0, outbuf1, outbuf2, outbuf3,
          row0buf, nzbuf0, nzbuf1, recbuf0, recbuf1,
          sem_tok0, sem_tok1, sem_ttl0, sem_ttl1,
          sem_idx0, sem_idx1, sem_out0, sem_out1, sem_out2, sem_out3):
    tokbuf = (tokbuf0, tokbuf1)
    tidx = (tidx0, tidx1)
    tokrows = (tokrows0, tokrows1)
    outbuf = (outbuf0, outbuf1, outbuf2, outbuf3)
    nzbuf = (nzbuf0, nzbuf1)
    recbuf = (recbuf0, recbuf1)
    sem_tok = (sem_tok0, sem_tok1)
    sem_ttl = (sem_ttl0, sem_ttl1)
    sem_idx = (sem_idx0, sem_idx1)
    sem_out = (sem_out0, sem_out1, sem_out2, sem_out3)

    wid = lax.axis_index("s") * 2 + lax.axis_index("c")
    base = wid * ROWS_PER_W
    iota = lax.iota(jnp.int32, LANES)

    pltpu.sync_copy(text_tab.at[pl.ds(0, 1)], row0buf)
    row0v = [row0buf[0, pl.ds(j * LANES, LANES)] for j in range(D // LANES)]

    def idx_copy(chunk, p):
        row0 = base + chunk * CHUNK
        pltpu.async_copy(tokens_hbm.at[pl.ds(row0 * L, CL)], tokbuf[p],
                         sem_idx[p])
        pltpu.async_copy(titles_hbm.at[pl.ds(row0, CHUNK)], tidx[p],
                         sem_idx[p])

    def idx_wait(chunk, p):
        row0 = base + chunk * CHUNK
        pltpu.make_async_copy(tokens_hbm.at[pl.ds(row0 * L, CL)], tokbuf[p],
                              sem_idx[p]).wait()
        pltpu.make_async_copy(titles_hbm.at[pl.ds(row0, CHUNK)], tidx[p],
                              sem_idx[p]).wait()

    def gather_issue(p, o):
        pltpu.async_copy(text_tab.at[tokbuf[p]], tokrows[p], sem_tok[p])
        pltpu.async_copy(title_tab.at[tidx[p]],
                         outbuf[o].at[:, pl.ds(0, D)], sem_ttl[p])

    def gather_wait(p, o):
        pltpu.make_async_copy(text_tab.at[tokbuf[p]], tokrows[p],
                              sem_tok[p]).wait()
        pltpu.make_async_copy(title_tab.at[tidx[p]],
                              outbuf[o].at[:, pl.ds(0, D)], sem_ttl[p]).wait()

    def out_issue(chunk, o):
        row0 = base + chunk * CHUNK
        pltpu.async_copy(outbuf[o], out_hbm.at[pl.ds(row0, CHUNK)],
                         sem_out[o])

    def out_wait(chunk, o):
        row0 = base + chunk * CHUNK
        pltpu.make_async_copy(outbuf[o], out_hbm.at[pl.ds(row0, CHUNK)],
                              sem_out[o]).wait()

    def counts(p):
        nz = jnp.zeros((LANES,), jnp.float32)
        for t in range(L):
            tv = plsc.load_gather(tokbuf[p], [iota * L + t])
            nz = nz + jnp.where(tv == 0, 1.0, 0.0)
        nzbuf[p][pl.ds(0, LANES)] = nz
        recbuf[p][pl.ds(0, LANES)] = \
            1.0 / jnp.maximum(jnp.float32(L) - nz, 1.0)

    def compute(p, o):
        @plsc.parallel_loop(0, CHUNK, unroll=UNROLL)
        def _(r):
            nzv = jnp.full((LANES,), nzbuf[p][pl.ds(r, LANES)][0],
                           jnp.float32)
            rec = jnp.full((LANES,), recbuf[p][pl.ds(r, LANES)][0],
                           jnp.float32)
            for j in range(D // LANES):
                vals = [tokrows[p][r * L + t, pl.ds(j * LANES, LANES)]
                        for t in range(L)]
                while len(vals) > 1:
                    nxt = [vals[k] + vals[k + 1]
                           for k in range(0, len(vals) - 1, 2)]
                    if len(vals) % 2:
                        nxt.append(vals[-1])
                    vals = nxt
                outbuf[o][r, pl.ds(D + j * LANES, LANES)] = \
                    (vals[0] - nzv * row0v[j]) * rec

    pltpu.sync_copy(tokens_hbm.at[pl.ds(base * L, CL)], tokbuf[0])
    pltpu.sync_copy(titles_hbm.at[pl.ds(base, CHUNK)], tidx[0])
    counts(0)
    gather_issue(0, 0)
    idx_copy(1, 1)

    def outer(i, _):
        for u in range(NOUT):
            c = i * NOUT + u
            p = u % 2
            o = u
            pprev = 1 - p

            gather_wait(p, o)

            @pl.when(c + 1 < N_CHUNKS)
            def _():
                idx_wait(c + 1, pprev)

                @pl.when(c >= 3)
                def _():
                    out_wait(c - 3, (u + 1) % NOUT)

                counts(pprev)
                gather_issue(pprev, (u + 1) % NOUT)

            compute(p, o)
            out_issue(c, o)

            @pl.when(c + 2 < N_CHUNKS)
            def _():
                idx_copy(c + 2, p)
        return 0

    lax.fori_loop(0, N_CHUNKS // NOUT, outer, 0)

    for k in range(NOUT):
        c = N_CHUNKS - NOUT + k
        out_wait(c, c % NOUT)


@functools.partial(jax.jit, static_argnums=())
def _sc_call(titles_i, tokens_i, title_table, text_table):
    mesh = plsc.VectorSubcoreMesh(core_axis_name="c", subcore_axis_name="s")
    return pl.kernel(
        _body,
        out_type=jax.ShapeDtypeStruct((B, D_OUT), jnp.float32),
        mesh=mesh,
        scratch_types=[
            pltpu.VMEM((CL,), jnp.int32),
            pltpu.VMEM((CL,), jnp.int32),
            pltpu.VMEM((CHUNK,), jnp.int32),
            pltpu.VMEM((CHUNK,), jnp.int32),
            pltpu.VMEM((CL, D), jnp.float32),
            pltpu.VMEM((CL, D), jnp.float32),
            pltpu.VMEM((CHUNK, D_OUT), jnp.float32),
            pltpu.VMEM((CHUNK, D_OUT), jnp.float32),
            pltpu.VMEM((CHUNK, D_OUT), jnp.float32),
            pltpu.VMEM((CHUNK, D_OUT), jnp.float32),
            pltpu.VMEM((1, D), jnp.float32),
            pltpu.VMEM((2 * LANES,), jnp.float32),
            pltpu.VMEM((2 * LANES,), jnp.float32),
            pltpu.VMEM((2 * LANES,), jnp.float32),
            pltpu.VMEM((2 * LANES,), jnp.float32),
            pltpu.SemaphoreType.DMA,
            pltpu.SemaphoreType.DMA,
            pltpu.SemaphoreType.DMA,
            pltpu.SemaphoreType.DMA,
            pltpu.SemaphoreType.DMA,
            pltpu.SemaphoreType.DMA,
            pltpu.SemaphoreType.DMA,
            pltpu.SemaphoreType.DMA,
            pltpu.SemaphoreType.DMA,
            pltpu.SemaphoreType.DMA,
        ],
        compiler_params=pltpu.CompilerParams(needs_layout_passes=False),
    )(titles_i, tokens_i, title_table, text_table)


def kernel(titles, tokens, title_table, text_table):
    titles_i = titles.astype(jnp.int32)
    tokens_i = tokens.reshape(-1).astype(jnp.int32)
    return _sc_call(titles_i, tokens_i, title_table, text_table)


# speedup vs baseline: 1.1669x; 1.1669x over previous
import functools

import jax
import jax.numpy as jnp
from jax import lax
from jax.experimental import pallas as pl
from jax.experimental.pallas import tpu as pltpu
from jax.experimental.pallas import tpu_sc as plsc

B = 16384
L = 20
D = 128
D_OUT = 2 * D

NUM_WORKERS = 32
ROWS_PER_W = B // NUM_WORKERS
CHUNK = 16
N_CHUNKS = ROWS_PER_W // CHUNK
LANES = 16
CL = CHUNK * L
NOUT = 4
UNROLL = 2


def _body(titles_hbm, tokens_hbm, title_tab, text_tab, out_hbm,
          tokbuf0, tokbuf1, tidx0, tidx1, tokrows0, tokrows1,
          outbuf0, outbuf1, outbuf2, outbuf3,
          row0buf, nzbuf0, nzbuf1, recbuf0, recbuf1,
          sem_tok0, sem_tok1, sem_ttl0, sem_ttl1,
          sem_idx0, sem_idx1, sem_out0, sem_out1, sem_out2, sem_out3):
    tokbuf = (tokbuf0, tokbuf1)
    tidx = (tidx0, tidx1)
    tokrows = (tokrows0, tokrows1)
    outbuf = (outbuf0, outbuf1, outbuf2, outbuf3)
    nzbuf = (nzbuf0, nzbuf1)
    recbuf = (recbuf0, recbuf1)
    sem_tok = (sem_tok0, sem_tok1)
    sem_ttl = (sem_ttl0, sem_ttl1)
    sem_idx = (sem_idx0, sem_idx1)
    sem_out = (sem_out0, sem_out1, sem_out2, sem_out3)

    wid = lax.axis_index("s") * 2 + lax.axis_index("c")
    base = wid * ROWS_PER_W
    iota = lax.iota(jnp.int32, LANES)

    pltpu.sync_copy(text_tab.at[pl.ds(0, 1)], row0buf)
    row0v = [row0buf[0, pl.ds(j * LANES, LANES)] for j in range(D // LANES)]

    def idx_copy(chunk, p):
        row0 = base + chunk * CHUNK
        pltpu.async_copy(tokens_hbm.at[pl.ds(row0 * L, CL)], tokbuf[p],
                         sem_idx[p])
        pltpu.async_copy(titles_hbm.at[pl.ds(row0, CHUNK)], tidx[p],
                         sem_idx[p])

    def idx_wait(chunk, p):
        row0 = base + chunk * CHUNK
        pltpu.make_async_copy(tokens_hbm.at[pl.ds(row0 * L, CL)], tokbuf[p],
                              sem_idx[p]).wait()
        pltpu.make_async_copy(titles_hbm.at[pl.ds(row0, CHUNK)], tidx[p],
                              sem_idx[p]).wait()

    def gather_issue(p, o):
        pltpu.async_copy(text_tab.at[tokbuf[p]], tokrows[p], sem_tok[p])
        pltpu.async_copy(title_tab.at[tidx[p]],
                         outbuf[o].at[:, pl.ds(0, D)], sem_ttl[p])

    def gather_wait(p, o):
        pltpu.make_async_copy(text_tab.at[tokbuf[p]], tokrows[p],
                              sem_tok[p]).wait()
        pltpu.make_async_copy(title_tab.at[tidx[p]],
                              outbuf[o].at[:, pl.ds(0, D)], sem_ttl[p]).wait()

    def out_issue(chunk, o):
        row0 = base + chunk * CHUNK
        pltpu.async_copy(outbuf[o], out_hbm.at[pl.ds(row0, CHUNK)],
                         sem_out[o])

    def out_wait(chunk, o):
        row0 = base + chunk * CHUNK
        pltpu.make_async_copy(outbuf[o], out_hbm.at[pl.ds(row0, CHUNK)],
                              sem_out[o]).wait()

    def counts(p):
        nz = jnp.zeros((LANES,), jnp.float32)
        for t in range(L):
            tv = plsc.load_gather(tokbuf[p], [iota * L + t])
            nz = nz + jnp.where(tv == 0, 1.0, 0.0)
        nzbuf[p][pl.ds(0, LANES)] = nz
        recbuf[p][pl.ds(0, LANES)] = \
            1.0 / jnp.maximum(jnp.float32(L) - nz, 1.0)

    def compute(p, o):
        @plsc.parallel_loop(0, CHUNK, unroll=UNROLL)
        def _(r):
            nzv = jnp.full((LANES,), nzbuf[p][pl.ds(r, LANES)][0],
                           jnp.float32)
            rec = jnp.full((LANES,), recbuf[p][pl.ds(r, LANES)][0],
                           jnp.float32)
            for j in range(D // LANES):
                vals = [tokrows[p][r * L + t, pl.ds(j * LANES, LANES)]
                        for t in range(L)]
                while len(vals) > 1:
                    nxt = [vals[k] + vals[k + 1]
                           for k in range(0, len(vals) - 1, 2)]
                    if len(vals) % 2:
                        nxt.append(vals[-1])
                    vals = nxt
                outbuf[o][r, pl.ds(D + j * LANES, LANES)] = \
                    (vals[0] - nzv * row0v[j]) * rec

    pltpu.sync_copy(tokens_hbm.at[pl.ds(base * L, CL)], tokbuf[0])
    pltpu.sync_copy(titles_hbm.at[pl.ds(base, CHUNK)], tidx[0])
    counts(0)
    gather_issue(0, 0)
    idx_copy(1, 1)

    def outer(i, _):
        for u in range(NOUT):
            c = i * NOUT + u
            p = u % 2
            o = u
            pprev = 1 - p

            gather_wait(p, o)

            @pl.when(c + 1 < N_CHUNKS)
            def _():
                idx_wait(c + 1, pprev)

                @pl.when(c >= 3)
                def _():
                    out_wait(c - 3, (u + 1) % NOUT)

                counts(pprev)
                gather_issue(pprev, (u + 1) % NOUT)

            compute(p, o)
            out_issue(c, o)

            @pl.when(c + 2 < N_CHUNKS)
            def _():
                idx_copy(c + 2, p)
        return 0

    lax.fori_loop(0, N_CHUNKS // NOUT, outer, 0)

    for k in range(NOUT):
        c = N_CHUNKS - NOUT + k
        out_wait(c, c % NOUT)


@functools.partial(jax.jit, static_argnums=())
def _sc_call(titles_i, tokens_i, title_table, text_table):
    mesh = plsc.VectorSubcoreMesh(core_axis_name="c", subcore_axis_name="s")
    return pl.kernel(
        _body,
        out_type=jax.ShapeDtypeStruct((B, D_OUT), jnp.float32),
        mesh=mesh,
        scratch_types=[
            pltpu.VMEM((CL,), jnp.int32),
            pltpu.VMEM((CL,), jnp.int32),
            pltpu.VMEM((CHUNK,), jnp.int32),
            pltpu.VMEM((CHUNK,), jnp.int32),
            pltpu.VMEM((CL, D), jnp.float32),
            pltpu.VMEM((CL, D), jnp.float32),
            pltpu.VMEM((CHUNK, D_OUT), jnp.float32),
            pltpu.VMEM((CHUNK, D_OUT), jnp.float32),
            pltpu.VMEM((CHUNK, D_OUT), jnp.float32),
            pltpu.VMEM((CHUNK, D_OUT), jnp.float32),
            pltpu.VMEM((1, D), jnp.float32),
            pltpu.VMEM((2 * LANES,), jnp.float32),
            pltpu.VMEM((2 * LANES,), jnp.float32),
            pltpu.VMEM((2 * LANES,), jnp.float32),
            pltpu.VMEM((2 * LANES,), jnp.float32),
            pltpu.SemaphoreType.DMA,
            pltpu.SemaphoreType.DMA,
            pltpu.SemaphoreType.DMA,
            pltpu.SemaphoreType.DMA,
            pltpu.SemaphoreType.DMA,
            pltpu.SemaphoreType.DMA,
            pltpu.SemaphoreType.DMA,
            pltpu.SemaphoreType.DMA,
            pltpu.SemaphoreType.DMA,
            pltpu.SemaphoreType.DMA,
        ],
        compiler_params=pltpu.CompilerParams(needs_layout_passes=False),
    )(titles_i, tokens_i, title_table, text_table)


def kernel(titles, tokens, title_table, text_table):
    titles_i = titles.astype(jnp.int32)
    tokens_i = tokens.reshape(-1).astype(jnp.int32)
    return _sc_call(titles_i, tokens_i, title_table, text_table)


# speedup vs baseline: 1.1705x; 1.0031x over previous
import functools

import jax
import jax.numpy as jnp
from jax import lax
from jax.experimental import pallas as pl
from jax.experimental.pallas import tpu as pltpu
from jax.experimental.pallas import tpu_sc as plsc

B = 16384
L = 20
D = 128
D_OUT = 2 * D

NUM_WORKERS = 32
ROWS_PER_W = B // NUM_WORKERS
CHUNK = 16
N_CHUNKS = ROWS_PER_W // CHUNK
LANES = 16
CL = CHUNK * L
NOUT = 4
UNROLL = 2


def _body(titles_hbm, tokens_hbm, title_tab, text_tab, out_hbm,
          tokbuf0, tokbuf1, tidx0, tidx1, tokrows0, tokrows1,
          outbuf0, outbuf1, outbuf2, outbuf3,
          row0buf, nzbuf0, nzbuf1, recbuf0, recbuf1,
          sem_tok0, sem_tok1, sem_ttl0, sem_ttl1,
          sem_idx0, sem_idx1, sem_out0, sem_out1, sem_out2, sem_out3):
    tokbuf = (tokbuf0, tokbuf1)
    tidx = (tidx0, tidx1)
    tokrows = (tokrows0, tokrows1)
    outbuf = (outbuf0, outbuf1, outbuf2, outbuf3)
    nzbuf = (nzbuf0, nzbuf1)
    recbuf = (recbuf0, recbuf1)
    sem_tok = (sem_tok0, sem_tok1)
    sem_ttl = (sem_ttl0, sem_ttl1)
    sem_idx = (sem_idx0, sem_idx1)
    sem_out = (sem_out0, sem_out1, sem_out2, sem_out3)

    wid = lax.axis_index("s") * 2 + lax.axis_index("c")
    base = wid * ROWS_PER_W
    iota = lax.iota(jnp.int32, LANES)

    pltpu.sync_copy(text_tab.at[pl.ds(0, 1)], row0buf)
    row0v = [row0buf[0, pl.ds(j * LANES, LANES)] for j in range(D // LANES)]

    def idx_copy(chunk, p):
        row0 = base + chunk * CHUNK
        pltpu.async_copy(tokens_hbm.at[pl.ds(row0 * L, CL)], tokbuf[p],
                         sem_idx[p])
        pltpu.async_copy(titles_hbm.at[pl.ds(row0, CHUNK)], tidx[p],
                         sem_idx[p])

    def idx_wait(chunk, p):
        row0 = base + chunk * CHUNK
        pltpu.make_async_copy(tokens_hbm.at[pl.ds(row0 * L, CL)], tokbuf[p],
                              sem_idx[p]).wait()
        pltpu.make_async_copy(titles_hbm.at[pl.ds(row0, CHUNK)], tidx[p],
                              sem_idx[p]).wait()

    def gather_issue(p, o):
        pltpu.async_copy(text_tab.at[tokbuf[p]], tokrows[p], sem_tok[p])
        pltpu.async_copy(title_tab.at[tidx[p]],
                         outbuf[o].at[:, pl.ds(0, D)], sem_ttl[p])

    def gather_wait(p, o):
        pltpu.make_async_copy(text_tab.at[tokbuf[p]], tokrows[p],
                              sem_tok[p]).wait()
        pltpu.make_async_copy(title_tab.at[tidx[p]],
                              outbuf[o].at[:, pl.ds(0, D)], sem_ttl[p]).wait()

    def out_issue(chunk, o):
        row0 = base + chunk * CHUNK
        pltpu.async_copy(outbuf[o], out_hbm.at[pl.ds(row0, CHUNK)],
                         sem_out[o])

    def out_wait(chunk, o):
        row0 = base + chunk * CHUNK
        pltpu.make_async_copy(outbuf[o], out_hbm.at[pl.ds(row0, CHUNK)],
                              sem_out[o]).wait()

    def counts(p):
        nz = jnp.zeros((LANES,), jnp.float32)
        for t in range(L):
            tv = plsc.load_gather(tokbuf[p], [iota * L + t])
            nz = nz + jnp.where(tv == 0, 1.0, 0.0)
        nzbuf[p][pl.ds(0, LANES)] = nz
        recbuf[p][pl.ds(0, LANES)] = \
            1.0 / jnp.maximum(jnp.float32(L) - nz, 1.0)

    def compute(p, o):
        @plsc.parallel_loop(0, CHUNK, unroll=UNROLL)
        def _(r):
            nzv = jnp.full((LANES,), nzbuf[p][pl.ds(r, LANES)][0],
                           jnp.float32)
            rec = jnp.full((LANES,), recbuf[p][pl.ds(r, LANES)][0],
                           jnp.float32)
            for j in range(D // LANES):
                vals = [tokrows[p][r * L + t, pl.ds(j * LANES, LANES)]
                        for t in range(L)]
                while len(vals) > 1:
                    nxt = [vals[k] + vals[k + 1]
                           for k in range(0, len(vals) - 1, 2)]
                    if len(vals) % 2:
                        nxt.append(vals[-1])
                    vals = nxt
                outbuf[o][r, pl.ds(D + j * LANES, LANES)] = \
                    (vals[0] - nzv * row0v[j]) * rec

    pltpu.sync_copy(tokens_hbm.at[pl.ds(base * L, CL)], tokbuf[0])
    pltpu.sync_copy(titles_hbm.at[pl.ds(base, CHUNK)], tidx[0])
    counts(0)
    gather_issue(0, 0)
    idx_copy(1, 1)

    def outer(i, _):
        for u in range(NOUT):
            c = i * NOUT + u
            p = u % 2
            o = u
            pprev = 1 - p

            gather_wait(p, o)

            @pl.when(c + 1 < N_CHUNKS)
            def _():
                idx_wait(c + 1, pprev)

                @pl.when(c >= 3)
                def _():
                    out_wait(c - 3, (u + 1) % NOUT)

                gather_issue(pprev, (u + 1) % NOUT)
                counts(pprev)

            compute(p, o)
            out_issue(c, o)

            @pl.when(c + 2 < N_CHUNKS)
            def _():
                idx_copy(c + 2, p)
        return 0

    lax.fori_loop(0, N_CHUNKS // NOUT, outer, 0)

    for k in range(NOUT):
        c = N_CHUNKS - NOUT + k
        out_wait(c, c % NOUT)


@functools.partial(jax.jit, static_argnums=())
def _sc_call(titles_i, tokens_i, title_table, text_table):
    mesh = plsc.VectorSubcoreMesh(core_axis_name="c", subcore_axis_name="s")
    return pl.kernel(
        _body,
        out_type=jax.ShapeDtypeStruct((B, D_OUT), jnp.float32),
        mesh=mesh,
        scratch_types=[
            pltpu.VMEM((CL,), jnp.int32),
            pltpu.VMEM((CL,), jnp.int32),
            pltpu.VMEM((CHUNK,), jnp.int32),
            pltpu.VMEM((CHUNK,), jnp.int32),
            pltpu.VMEM((CL, D), jnp.float32),
            pltpu.VMEM((CL, D), jnp.float32),
            pltpu.VMEM((CHUNK, D_OUT), jnp.float32),
            pltpu.VMEM((CHUNK, D_OUT), jnp.float32),
            pltpu.VMEM((CHUNK, D_OUT), jnp.float32),
            pltpu.VMEM((CHUNK, D_OUT), jnp.float32),
            pltpu.VMEM((1, D), jnp.float32),
            pltpu.VMEM((2 * LANES,), jnp.float32),
            pltpu.VMEM((2 * LANES,), jnp.float32),
            pltpu.VMEM((2 * LANES,), jnp.float32),
            pltpu.VMEM((2 * LANES,), jnp.float32),
            pltpu.SemaphoreType.DMA,
            pltpu.SemaphoreType.DMA,
            pltpu.SemaphoreType.DMA,
            pltpu.SemaphoreType.DMA,
            pltpu.SemaphoreType.DMA,
            pltpu.SemaphoreType.DMA,
            pltpu.SemaphoreType.DMA,
            pltpu.SemaphoreType.DMA,
            pltpu.SemaphoreType.DMA,
            pltpu.SemaphoreType.DMA,
        ],
        compiler_params=pltpu.CompilerParams(needs_layout_passes=False),
    )(titles_i, tokens_i, title_table, text_table)


def kernel(titles, tokens, title_table, text_table):
    titles_i = titles.astype(jnp.int32)
    tokens_i = tokens.reshape(-1).astype(jnp.int32)
    return _sc_call(titles_i, tokens_i, title_table, text_table)
